# Initial kernel scaffold; baseline (speedup 1.0000x reference)
import jax
import jax.numpy as jnp
from jax.experimental import pallas as pl


def kernel(x, edge_index, W1_0, b1_0, W2_0, b2_0, W1_1, b1_1, W2_1, b2_1, W1_2, b1_2, W2_2, b2_2, Wlin, blin):
    raise NotImplementedError("write your pallas kernel here")


# speedup vs baseline: 6.2616x; 6.2616x over previous
import functools

import jax
import jax.numpy as jnp
from jax import lax
from jax.experimental import pallas as pl
from jax.experimental.pallas import tpu as pltpu
from jax.experimental.pallas import tpu_sc as plsc

N = 10000
E = 320000
D = 128

NC = 2
NS = 16
CHUNK = 128
NCHUNKS = E // CHUNK
ROWS_MAIN = 624
TAIL = 16
ZROWS = 208


def _sc_aggregate_body(src_hbm, dst_hbm, h_hbm, out_hbm,
                       src_v, dst_v, rows_v, zero_v, agg_sh, sem):
    c = lax.axis_index("c")
    s = lax.axis_index("s")
    wid = c * NS + s

    zvec = jnp.zeros((16,), jnp.float32)

    @pl.loop(0, ZROWS)
    def _zero_fill(i):
        for j in range(D // 16):
            zero_v[i, pl.ds(j * 16, 16)] = zvec

    @pl.loop(0, ROWS_MAIN // ZROWS)
    def _zero_agg(j):
        pltpu.sync_copy(zero_v, agg_sh.at[pl.ds(s * ROWS_MAIN + j * ZROWS, ZROWS)])

    @pl.when(s == NS - 1)
    def _zero_tail():
        pltpu.sync_copy(zero_v.at[pl.ds(0, TAIL)], agg_sh.at[pl.ds(NS * ROWS_MAIN, TAIL)])

    plsc.subcore_barrier()

    @pl.loop(wid, NCHUNKS, step=NC * NS)
    def _edges(g):
        off = g * CHUNK
        pltpu.sync_copy(src_hbm.at[pl.ds(off, CHUNK)], src_v)
        pltpu.sync_copy(dst_hbm.at[pl.ds(off, CHUNK)], dst_v)
        pltpu.async_copy(h_hbm.at[src_v], rows_v, sem).wait()
        pltpu.sync_copy(rows_v, agg_sh.at[dst_v], add=True)

    plsc.subcore_barrier()

    pltpu.sync_copy(agg_sh.at[pl.ds(s * ROWS_MAIN, ROWS_MAIN)],
                    out_hbm.at[c, pl.ds(s * ROWS_MAIN, ROWS_MAIN)])

    @pl.when(s == NS - 1)
    def _flush_tail():
        pltpu.sync_copy(agg_sh.at[pl.ds(NS * ROWS_MAIN, TAIL)],
                        out_hbm.at[c, pl.ds(NS * ROWS_MAIN, TAIL)])


@jax.jit
def _sc_aggregate(src, dst, h):
    mesh = plsc.VectorSubcoreMesh(core_axis_name="c", subcore_axis_name="s")
    return pl.kernel(
        _sc_aggregate_body,
        out_type=jax.ShapeDtypeStruct((NC, N, D), jnp.float32),
        mesh=mesh,
        scratch_types=[
            pltpu.VMEM((CHUNK,), jnp.int32),
            pltpu.VMEM((CHUNK,), jnp.int32),
            pltpu.VMEM((CHUNK, D), jnp.float32),
            pltpu.VMEM((ZROWS, D), jnp.float32),
            pltpu.VMEM_SHARED((N, D), jnp.float32),
            pltpu.SemaphoreType.DMA,
        ],
    )(src, dst, h)


BN = 1000


def _mlp_body(h_ref, a_ref, w1_ref, b1_ref, w2_ref, b2_ref, out_ref):
    t = h_ref[...] + a_ref[0] + a_ref[1]
    t = jnp.maximum(jnp.dot(t, w1_ref[...], preferred_element_type=jnp.float32)
                    + b1_ref[...], 0.0)
    t = jnp.dot(t, w2_ref[...], preferred_element_type=jnp.float32) + b2_ref[...]
    out_ref[...] = jnp.maximum(t, 0.0)


def _mlp_final_body(h_ref, a_ref, w1_ref, b1_ref, w2_ref, b2_ref,
                    wl_ref, bl_ref, out_ref):
    t = h_ref[...] + a_ref[0] + a_ref[1]
    t = jnp.maximum(jnp.dot(t, w1_ref[...], preferred_element_type=jnp.float32)
                    + b1_ref[...], 0.0)
    t = jnp.dot(t, w2_ref[...], preferred_element_type=jnp.float32) + b2_ref[...]
    t = jnp.maximum(t, 0.0)
    out_ref[...] = jnp.dot(t, wl_ref[...], preferred_element_type=jnp.float32) + bl_ref[...]


_row_spec = pl.BlockSpec((BN, D), lambda i: (i, 0))
_agg_spec = pl.BlockSpec((NC, BN, D), lambda i: (0, i, 0))
_w_spec = pl.BlockSpec((D, D), lambda i: (0, 0))
_b_spec = pl.BlockSpec((1, D), lambda i: (0, 0))


@jax.jit
def _mlp(h, agg, w1, b1, w2, b2):
    return pl.pallas_call(
        _mlp_body,
        grid=(N // BN,),
        in_specs=[_row_spec, _agg_spec, _w_spec, _b_spec, _w_spec, _b_spec],
        out_specs=_row_spec,
        out_shape=jax.ShapeDtypeStruct((N, D), jnp.float32),
    )(h, agg, w1, b1.reshape(1, D), w2, b2.reshape(1, D))


@jax.jit
def _mlp_final(h, agg, w1, b1, w2, b2, wl, bl):
    return pl.pallas_call(
        _mlp_final_body,
        grid=(N // BN,),
        in_specs=[_row_spec, _agg_spec, _w_spec, _b_spec, _w_spec, _b_spec,
                  _w_spec, _b_spec],
        out_specs=_row_spec,
        out_shape=jax.ShapeDtypeStruct((N, D), jnp.float32),
    )(h, agg, w1, b1.reshape(1, D), w2, b2.reshape(1, D),
      wl, bl.reshape(1, D))


def kernel(x, edge_index, W1_0, b1_0, W2_0, b2_0, W1_1, b1_1, W2_1, b2_1,
           W1_2, b1_2, W2_2, b2_2, Wlin, blin):
    src = edge_index[0]
    dst = edge_index[1]
    agg0 = _sc_aggregate(src, dst, x)
    h1 = _mlp(x, agg0, W1_0, b1_0, W2_0, b2_0)
    agg1 = _sc_aggregate(src, dst, h1)
    h2 = _mlp(h1, agg1, W1_1, b1_1, W2_1, b2_1)
    agg2 = _sc_aggregate(src, dst, h2)
    return _mlp_final(h2, agg2, W1_2, b1_2, W2_2, b2_2, Wlin, blin)
